# baseline (speedup 1.0000x reference)
import functools

import jax
import jax.numpy as jnp
import numpy as np
from jax import lax
from jax.experimental import pallas as pl
from jax.experimental.pallas import tpu as pltpu
from jax.experimental.pallas import tpu_sc as plsc

BATCH = 16384
NUM_FEATURES = 26
FEATURE_DIM = 28
OUT_DIM = 32

NUM_WORKERS = 32
ROWS_PER_WORKER = BATCH // NUM_WORKERS
CHUNK_ROWS = 32
NUM_CHUNKS = ROWS_PER_WORKER // CHUNK_ROWS
CFLAT = CHUNK_ROWS * NUM_FEATURES
GATHER_SLICE = 64
NUM_GATHERS = CFLAT // GATHER_SLICE
CODE_COLS = 64
CROWS = CFLAT // CODE_COLS
OUT_TILE_ROWS = CFLAT * OUT_DIM // 128

_FOFF = ((np.arange(CFLAT, dtype=np.int64) % NUM_FEATURES) * 100000 + 1).astype(
    np.int32
).reshape(CROWS, CODE_COLS)


def _sc_body(feat_hbm, codes_hbm, tmpl_hbm, foff_hbm, out_hbm,
             codes_v, idxa_v, idxb_v, foff_v, w_v, rowsa_v, rowsb_v,
             out_tile, tmpl_v, sem):
    wid = lax.axis_index("s") * 2 + lax.axis_index("c")

    pltpu.sync_copy(foff_hbm, foff_v)
    pltpu.sync_copy(tmpl_hbm, tmpl_v)

    lane = lax.iota(jnp.int32, 16)
    head_mask = lane < 4
    cidx0 = jnp.maximum(lane - 4, 0)
    cidx1 = lane + 12

    def chunk_body(ch, _):
        croff = wid * (NUM_CHUNKS * CROWS) + ch * CROWS
        pltpu.sync_copy(codes_hbm.at[pl.ds(croff, CROWS)], codes_v)

        def idx_body(q, _):
            for k in range(CODE_COLS // 16):
                s = k * 16
                c = codes_v[q, pl.ds(s, 16)]
                idx = c.astype(jnp.int32) + foff_v[q, pl.ds(s, 16)]
                d7 = idx // 7
                ka = idx + d7
                idxa_v[q, pl.ds(s, 16)] = ka
                idxb_v[q, pl.ds(s, 16)] = ka + 1
                w_v[q, pl.ds(s, 16)] = (idx - d7 * 7) * 4
            return 0

        lax.fori_loop(0, CROWS, idx_body, 0)

        gathers = []
        for j in range(NUM_GATHERS):
            gathers.append(pltpu.async_copy(
                feat_hbm.at[idxa_v.at[j]],
                rowsa_v.at[pl.ds(j * GATHER_SLICE, GATHER_SLICE)], sem))
            gathers.append(pltpu.async_copy(
                feat_hbm.at[idxb_v.at[j]],
                rowsb_v.at[pl.ds(j * GATHER_SLICE, GATHER_SLICE)], sem))
        for g in gathers:
            g.wait()

        def repack_body(q, _):
            for k in range(4):
                r = q * 4 + k
                f = lax.rem(r, NUM_FEATURES)
                t = tmpl_v[f, :]
                rsplat = jnp.full((16,), r, jnp.int32)
                wsplat = plsc.load_gather(
                    w_v, [jnp.full((16,), r // CODE_COLS, jnp.int32),
                          jnp.full((16,), lax.rem(r, CODE_COLS), jnp.int32)])

                base28 = jnp.full(
                    (16,), r * 28 + 256 * (r // GATHER_SLICE), jnp.int32)

                s0 = wsplat + cidx0
                a0 = base28 + s0
                b0 = jnp.maximum(a0 - 28, 0)
                ga = plsc.load_gather(rowsa_v, [a0 >> 5, a0 & 31])
                gb = plsc.load_gather(rowsb_v, [b0 >> 5, b0 & 31])
                ve = jnp.where(head_mask, t, jnp.where(s0 < 28, ga, gb))

                s1 = wsplat + cidx1
                a1 = base28 + s1
                b1 = jnp.maximum(a1 - 28, 0)
                ga1 = plsc.load_gather(rowsa_v, [a1 >> 5, a1 & 31])
                gb1 = plsc.load_gather(rowsb_v, [b1 >> 5, b1 & 31])
                vo = jnp.where(s1 < 28, ga1, gb1)

                out_tile[q, pl.ds(k * 32, 16)] = ve
                out_tile[q, pl.ds(k * 32 + 16, 16)] = vo
            return 0

        lax.fori_loop(0, OUT_TILE_ROWS, repack_body, 0)

        oroff = wid * (NUM_CHUNKS * OUT_TILE_ROWS) + ch * OUT_TILE_ROWS
        pltpu.sync_copy(out_tile, out_hbm.at[pl.ds(oroff, OUT_TILE_ROWS)])
        return 0

    lax.fori_loop(0, NUM_CHUNKS, chunk_body, 0)


@jax.jit
def _run(feature_embedding, codes_2d, tmpl, foff):
    mesh = plsc.VectorSubcoreMesh(core_axis_name="c", subcore_axis_name="s")
    k = functools.partial(
        pl.kernel,
        mesh=mesh,
        out_type=jax.ShapeDtypeStruct((BATCH * NUM_FEATURES * OUT_DIM // 128, 128),
                                      jnp.float32),
        scratch_types=[
            pltpu.VMEM((CROWS, CODE_COLS), jnp.float32),
            pltpu.VMEM((CROWS, CODE_COLS), jnp.int32),
            pltpu.VMEM((CROWS, CODE_COLS), jnp.int32),
            pltpu.VMEM((CROWS, CODE_COLS), jnp.int32),
            pltpu.VMEM((CROWS, CODE_COLS), jnp.int32),
            pltpu.VMEM((CFLAT, FEATURE_DIM), jnp.float32),
            pltpu.VMEM((CFLAT, FEATURE_DIM), jnp.float32),
            pltpu.VMEM((OUT_TILE_ROWS, 128), jnp.float32),
            pltpu.VMEM((NUM_FEATURES, 16), jnp.float32),
            pltpu.SemaphoreType.DMA,
        ],
        compiler_params=pltpu.CompilerParams(use_tc_tiling_on_sc=False,
                                             needs_layout_passes=False),
    )(_sc_body)
    return k(feature_embedding, codes_2d, tmpl, foff)


def kernel(inputs, feature_embedding, shared_embedding):
    codes_2d = inputs.reshape(BATCH * NUM_FEATURES // CODE_COLS, CODE_COLS)
    tmpl = jnp.pad(shared_embedding, ((0, 0), (0, 12)))
    out = _run(feature_embedding, codes_2d, tmpl, jnp.asarray(_FOFF))
    return out.reshape(BATCH, NUM_FEATURES, OUT_DIM)


# speedup vs baseline: 1.3708x; 1.3708x over previous
import functools

import jax
import jax.numpy as jnp
import numpy as np
from jax import lax
from jax.experimental import pallas as pl
from jax.experimental.pallas import tpu as pltpu
from jax.experimental.pallas import tpu_sc as plsc

BATCH = 16384
NUM_FEATURES = 26
FEATURE_DIM = 28
OUT_DIM = 32
NUM_ROWS = 2600001
NUM_ROWS_PAD = 2600004

NUM_WORKERS = 32
ROWS_PER_WORKER = BATCH // NUM_WORKERS
CHUNK_ROWS = 64
NUM_CHUNKS = ROWS_PER_WORKER // CHUNK_ROWS
CFLAT = CHUNK_ROWS * NUM_FEATURES
GATHER_SLICE = 128
NUM_GATHERS = CFLAT // GATHER_SLICE
CROWS = CFLAT // 128
OUT_TILE_ROWS = CFLAT * OUT_DIM // 128

PREP_BLOCK_ROWS = 8192
PREP_GRID = -(-NUM_ROWS // PREP_BLOCK_ROWS)

_FOFF = ((np.arange(CFLAT, dtype=np.int64) % NUM_FEATURES) * 100000 + 1).astype(
    np.int32
).reshape(CROWS, 128)


def _prep_body(in_ref, out_ref):
    x = in_ref[...]
    x4 = x.reshape(PREP_BLOCK_ROWS // 4, 4, FEATURE_DIM)
    parts = [
        jnp.pad(x4[:, k, :], ((0, 0), (0, OUT_DIM - FEATURE_DIM)))
        for k in range(4)
    ]
    out_ref[...] = jnp.concatenate(parts, axis=1)


@jax.jit
def _prep(feature_embedding):
    return pl.pallas_call(
        _prep_body,
        grid=(PREP_GRID,),
        in_specs=[pl.BlockSpec((PREP_BLOCK_ROWS, FEATURE_DIM), lambda i: (i, 0))],
        out_specs=pl.BlockSpec((PREP_BLOCK_ROWS // 4, 128), lambda i: (i, 0)),
        out_shape=jax.ShapeDtypeStruct((NUM_ROWS_PAD * OUT_DIM // 128, 128),
                                       jnp.float32),
    )(feature_embedding)


def _sc_body(feat_hbm, codes_hbm, tmpl_hbm, foff_hbm, out_hbm,
             codes_v, idx_v, foff_v, rows_v, out_tile, tmpl_v, sem):
    wid = lax.axis_index("s") * 2 + lax.axis_index("c")

    pltpu.sync_copy(foff_hbm, foff_v)
    pltpu.sync_copy(tmpl_hbm, tmpl_v)

    lane = lax.iota(jnp.int32, 16)
    head_mask = lane < 4
    cidx0 = jnp.maximum(lane - 4, 0)
    cidx1 = lane + 12

    def chunk_body(ch, _):
        croff = wid * (NUM_CHUNKS * CROWS) + ch * CROWS
        pltpu.sync_copy(codes_hbm.at[pl.ds(croff, CROWS)], codes_v)

        def idx_body(q, _):
            for k in range(8):
                s = k * 16
                c = codes_v[q, pl.ds(s, 16)]
                idx_v[q, pl.ds(s, 16)] = c.astype(jnp.int32) + foff_v[q, pl.ds(s, 16)]
            return 0

        lax.fori_loop(0, CROWS, idx_body, 0)

        gathers = [
            pltpu.async_copy(
                feat_hbm.at[idx_v.at[j]],
                rows_v.at[pl.ds(j * GATHER_SLICE, GATHER_SLICE)], sem)
            for j in range(NUM_GATHERS)
        ]
        for g in gathers:
            g.wait()

        def repack_body(q, _):
            for k in range(4):
                r = q * 4 + k
                f = lax.rem(r, NUM_FEATURES)
                t = tmpl_v[f, :]
                rsplat = jnp.full((16,), r, jnp.int32)
                g0 = plsc.load_gather(rows_v, [rsplat, cidx0])
                ve = jnp.where(head_mask, t, g0)
                vo = plsc.load_gather(rows_v, [rsplat, cidx1])
                out_tile[q, pl.ds(k * 32, 16)] = ve
                out_tile[q, pl.ds(k * 32 + 16, 16)] = vo
            return 0

        lax.fori_loop(0, OUT_TILE_ROWS, repack_body, 0)

        oroff = wid * (NUM_CHUNKS * OUT_TILE_ROWS) + ch * OUT_TILE_ROWS
        pltpu.sync_copy(out_tile, out_hbm.at[pl.ds(oroff, OUT_TILE_ROWS)])
        return 0

    lax.fori_loop(0, NUM_CHUNKS, chunk_body, 0)


@jax.jit
def _run(feature_embedding, codes_2d, tmpl, foff):
    mesh = plsc.VectorSubcoreMesh(core_axis_name="c", subcore_axis_name="s")
    k = functools.partial(
        pl.kernel,
        mesh=mesh,
        out_type=jax.ShapeDtypeStruct((BATCH * NUM_FEATURES * OUT_DIM // 128, 128),
                                      jnp.float32),
        scratch_types=[
            pltpu.VMEM((CROWS, 128), jnp.float32),
            pltpu.VMEM((CROWS, 128), jnp.int32),
            pltpu.VMEM((CROWS, 128), jnp.int32),
            pltpu.VMEM((CFLAT, OUT_DIM), jnp.float32),
            pltpu.VMEM((OUT_TILE_ROWS, 128), jnp.float32),
            pltpu.VMEM((NUM_FEATURES, 16), jnp.float32),
            pltpu.SemaphoreType.DMA,
        ],
        compiler_params=pltpu.CompilerParams(use_tc_tiling_on_sc=False,
                                             needs_layout_passes=False),
    )(_sc_body)
    return k(feature_embedding, codes_2d, tmpl, foff)


def kernel(inputs, feature_embedding, shared_embedding):
    table32 = _prep(feature_embedding).reshape(NUM_ROWS_PAD, OUT_DIM)
    codes_2d = inputs.reshape(BATCH * NUM_FEATURES // 128, 128)
    tmpl = jnp.pad(shared_embedding, ((0, 0), (0, 12)))
    out = _run(table32, codes_2d, tmpl, jnp.asarray(_FOFF))
    return out.reshape(BATCH, NUM_FEATURES, OUT_DIM)


# speedup vs baseline: 1.3722x; 1.0010x over previous
import functools

import jax
import jax.numpy as jnp
import numpy as np
from jax import lax
from jax.experimental import pallas as pl
from jax.experimental.pallas import tpu as pltpu
from jax.experimental.pallas import tpu_sc as plsc

BATCH = 16384
NUM_FEATURES = 26
FEATURE_DIM = 28
OUT_DIM = 32
NUM_ROWS = 2600001
NUM_ROWS_PAD = 2600004

NUM_WORKERS = 32
ROWS_PER_WORKER = BATCH // NUM_WORKERS
CHUNK_ROWS = 64
NUM_CHUNKS = ROWS_PER_WORKER // CHUNK_ROWS
CFLAT = CHUNK_ROWS * NUM_FEATURES
GATHER_SLICE = 128
NUM_GATHERS = CFLAT // GATHER_SLICE
CROWS = CFLAT // 128
OUT_TILE_ROWS = CFLAT * OUT_DIM // 128

PREP_BLOCK_ROWS = 8192
PREP_GRID = -(-NUM_ROWS // PREP_BLOCK_ROWS)

_FOFF = ((np.arange(CFLAT, dtype=np.int64) % NUM_FEATURES) * 100000 + 1).astype(
    np.int32
).reshape(CROWS, 128)


def _prep_body(in_ref, out_ref):
    x = in_ref[...]
    x4 = x.reshape(PREP_BLOCK_ROWS // 4, 4, FEATURE_DIM)
    parts = [
        jnp.pad(x4[:, k, :], ((0, 0), (0, OUT_DIM - FEATURE_DIM)))
        for k in range(4)
    ]
    out_ref[...] = jnp.concatenate(parts, axis=1)


@jax.jit
def _prep(feature_embedding):
    return pl.pallas_call(
        _prep_body,
        grid=(PREP_GRID,),
        in_specs=[pl.BlockSpec((PREP_BLOCK_ROWS, FEATURE_DIM), lambda i: (i, 0))],
        out_specs=pl.BlockSpec((PREP_BLOCK_ROWS // 4, 128), lambda i: (i, 0)),
        out_shape=jax.ShapeDtypeStruct((NUM_ROWS_PAD * OUT_DIM // 128, 128),
                                       jnp.float32),
    )(feature_embedding)


def _sc_body(feat_hbm, codes_hbm, tmpl_hbm, foff_hbm, out_hbm,
             codes_v, idx_v, foff_v, rows_v, out_tile, tmpl_v, sem):
    wid = lax.axis_index("s") * 2 + lax.axis_index("c")

    pltpu.sync_copy(foff_hbm, foff_v)
    pltpu.sync_copy(tmpl_hbm, tmpl_v)

    lane = lax.iota(jnp.int32, 16)
    head_mask = lane < 4
    cidx0 = jnp.maximum(lane - 4, 0)
    cidx1 = lane + 12

    def chunk_body(ch, _):
        croff = wid * (NUM_CHUNKS * CROWS) + ch * CROWS
        pltpu.sync_copy(codes_hbm.at[pl.ds(croff, CROWS)], codes_v)

        def idx_body(q, _):
            for k in range(8):
                s = k * 16
                c = codes_v[q, pl.ds(s, 16)]
                idx_v[q, pl.ds(s, 16)] = c.astype(jnp.int32) + foff_v[q, pl.ds(s, 16)]
            return 0

        lax.fori_loop(0, CROWS, idx_body, 0)

        gathers = [
            pltpu.async_copy(
                feat_hbm.at[idx_v.at[j]],
                rows_v.at[pl.ds(j * GATHER_SLICE, GATHER_SLICE)], sem)
            for j in range(NUM_GATHERS)
        ]
        for g in gathers:
            g.wait()

        def repack_body(r, _):
            b = r // NUM_FEATURES
            f = lax.rem(r, NUM_FEATURES)
            t = tmpl_v[f, :]
            rsplat = jnp.full((16,), r, jnp.int32)
            g0 = plsc.load_gather(rows_v, [rsplat, cidx0])
            ve = jnp.where(head_mask, t, g0)
            vo = plsc.load_gather(rows_v, [rsplat, cidx1])
            out_tile[b, f, pl.ds(0, 16)] = ve
            out_tile[b, f, pl.ds(16, 16)] = vo
            return 0

        lax.fori_loop(0, CFLAT, repack_body, 0)

        boff = wid * (NUM_CHUNKS * CHUNK_ROWS) + ch * CHUNK_ROWS
        pltpu.sync_copy(out_tile, out_hbm.at[pl.ds(boff, CHUNK_ROWS)])
        return 0

    lax.fori_loop(0, NUM_CHUNKS, chunk_body, 0)


@jax.jit
def _run(feature_embedding, codes_2d, tmpl, foff):
    mesh = plsc.VectorSubcoreMesh(core_axis_name="c", subcore_axis_name="s")
    k = functools.partial(
        pl.kernel,
        mesh=mesh,
        out_type=jax.ShapeDtypeStruct((BATCH, NUM_FEATURES, OUT_DIM), jnp.float32),
        scratch_types=[
            pltpu.VMEM((CROWS, 128), jnp.float32),
            pltpu.VMEM((CROWS, 128), jnp.int32),
            pltpu.VMEM((CROWS, 128), jnp.int32),
            pltpu.VMEM((CFLAT, OUT_DIM), jnp.float32),
            pltpu.VMEM((CHUNK_ROWS, NUM_FEATURES, OUT_DIM), jnp.float32),
            pltpu.VMEM((NUM_FEATURES, 16), jnp.float32),
            pltpu.SemaphoreType.DMA,
        ],
        compiler_params=pltpu.CompilerParams(use_tc_tiling_on_sc=False,
                                             needs_layout_passes=False),
    )(_sc_body)
    return k(feature_embedding, codes_2d, tmpl, foff)


def kernel(inputs, feature_embedding, shared_embedding):
    table32 = _prep(feature_embedding).reshape(NUM_ROWS_PAD, OUT_DIM)
    codes_2d = inputs.reshape(BATCH * NUM_FEATURES // 128, 128)
    tmpl = jnp.pad(shared_embedding, ((0, 0), (0, 12)))
    return _run(table32, codes_2d, tmpl, jnp.asarray(_FOFF))


# speedup vs baseline: 1.4174x; 1.0329x over previous
import functools

import jax
import jax.numpy as jnp
import numpy as np
from jax import lax
from jax.experimental import pallas as pl
from jax.experimental.pallas import tpu as pltpu
from jax.experimental.pallas import tpu_sc as plsc

BATCH = 16384
NUM_FEATURES = 26
FEATURE_DIM = 28
OUT_DIM = 32
NUM_ROWS = 2600001
NUM_ROWS_PAD = 2600004

NUM_WORKERS = 32
ROWS_PER_WORKER = BATCH // NUM_WORKERS
CHUNK_ROWS = 64
NUM_CHUNKS = ROWS_PER_WORKER // CHUNK_ROWS
CFLAT = CHUNK_ROWS * NUM_FEATURES
GATHER_SLICE = 128
NUM_GATHERS = CFLAT // GATHER_SLICE
CROWS = CFLAT // 128
OUT_TILE_ROWS = CFLAT * OUT_DIM // 128

PREP_BLOCK_ROWS = 16384
PREP_GRID = -(-NUM_ROWS // PREP_BLOCK_ROWS)

_FOFF = ((np.arange(CFLAT, dtype=np.int64) % NUM_FEATURES) * 100000 + 1).astype(
    np.int32
).reshape(CROWS, 128)


def _prep_body(in_ref, out_ref):
    x = in_ref[...]
    x4 = x.reshape(PREP_BLOCK_ROWS // 4, 4, FEATURE_DIM)
    parts = [
        jnp.pad(x4[:, k, :], ((0, 0), (0, OUT_DIM - FEATURE_DIM)))
        for k in range(4)
    ]
    out_ref[...] = jnp.concatenate(parts, axis=1)


@jax.jit
def _prep(feature_embedding):
    return pl.pallas_call(
        _prep_body,
        grid=(PREP_GRID,),
        in_specs=[pl.BlockSpec((PREP_BLOCK_ROWS, FEATURE_DIM), lambda i: (i, 0))],
        out_specs=pl.BlockSpec((PREP_BLOCK_ROWS // 4, 128), lambda i: (i, 0)),
        out_shape=jax.ShapeDtypeStruct((NUM_ROWS_PAD * OUT_DIM // 128, 128),
                                       jnp.float32),
    )(feature_embedding)


def _sc_body(feat_hbm, codes_hbm, tmpl_hbm, foff_hbm, out_hbm,
             codes_v, idx_v, foff_v, rows_v, out_tile, tmpl_v, sem):
    wid = lax.axis_index("s") * 2 + lax.axis_index("c")

    pltpu.sync_copy(foff_hbm, foff_v)
    pltpu.sync_copy(tmpl_hbm, tmpl_v)

    lane = lax.iota(jnp.int32, 16)
    head_mask = lane < 4
    cidx0 = jnp.maximum(lane - 4, 0)
    cidx1 = lane + 12

    def chunk_body(ch, _):
        croff = wid * (NUM_CHUNKS * CROWS) + ch * CROWS
        pltpu.sync_copy(codes_hbm.at[pl.ds(croff, CROWS)], codes_v)

        def idx_body(q, _):
            for k in range(8):
                s = k * 16
                c = codes_v[q, pl.ds(s, 16)]
                idx_v[q, pl.ds(s, 16)] = c.astype(jnp.int32) + foff_v[q, pl.ds(s, 16)]
            return 0

        lax.fori_loop(0, CROWS, idx_body, 0)

        gathers = [
            pltpu.async_copy(
                feat_hbm.at[idx_v.at[j]],
                rows_v.at[pl.ds(j * GATHER_SLICE, GATHER_SLICE)], sem)
            for j in range(NUM_GATHERS)
        ]
        for g in gathers:
            g.wait()

        def repack_body(r, _):
            b = r // NUM_FEATURES
            f = lax.rem(r, NUM_FEATURES)
            t = tmpl_v[f, :]
            rsplat = jnp.full((16,), r, jnp.int32)
            g0 = plsc.load_gather(rows_v, [rsplat, cidx0])
            ve = jnp.where(head_mask, t, g0)
            vo = plsc.load_gather(rows_v, [rsplat, cidx1])
            out_tile[b, f, pl.ds(0, 16)] = ve
            out_tile[b, f, pl.ds(16, 16)] = vo
            return 0

        lax.fori_loop(0, CFLAT, repack_body, 0)

        boff = wid * (NUM_CHUNKS * CHUNK_ROWS) + ch * CHUNK_ROWS
        pltpu.sync_copy(out_tile, out_hbm.at[pl.ds(boff, CHUNK_ROWS)])
        return 0

    lax.fori_loop(0, NUM_CHUNKS, chunk_body, 0)


@jax.jit
def _run(feature_embedding, codes_2d, tmpl, foff):
    mesh = plsc.VectorSubcoreMesh(core_axis_name="c", subcore_axis_name="s")
    k = functools.partial(
        pl.kernel,
        mesh=mesh,
        out_type=jax.ShapeDtypeStruct((BATCH, NUM_FEATURES, OUT_DIM), jnp.float32),
        scratch_types=[
            pltpu.VMEM((CROWS, 128), jnp.float32),
            pltpu.VMEM((CROWS, 128), jnp.int32),
            pltpu.VMEM((CROWS, 128), jnp.int32),
            pltpu.VMEM((CFLAT, OUT_DIM), jnp.float32),
            pltpu.VMEM((CHUNK_ROWS, NUM_FEATURES, OUT_DIM), jnp.float32),
            pltpu.VMEM((NUM_FEATURES, 16), jnp.float32),
            pltpu.SemaphoreType.DMA,
        ],
        compiler_params=pltpu.CompilerParams(use_tc_tiling_on_sc=False,
                                             needs_layout_passes=False),
    )(_sc_body)
    return k(feature_embedding, codes_2d, tmpl, foff)


def kernel(inputs, feature_embedding, shared_embedding):
    table32 = _prep(feature_embedding).reshape(NUM_ROWS_PAD, OUT_DIM)
    codes_2d = inputs.reshape(BATCH * NUM_FEATURES // 128, 128)
    tmpl = jnp.pad(shared_embedding, ((0, 0), (0, 12)))
    return _run(table32, codes_2d, tmpl, jnp.asarray(_FOFF))
